# baseline (speedup 1.0000x reference)
import functools

import jax
import jax.numpy as jnp
from jax import lax
from jax.experimental import pallas as pl
from jax.experimental.pallas import tpu as pltpu
from jax.experimental.pallas import tpu_sc as plsc

NC = 2
NS = 16
NT = NC * NS
CHUNK = 125


def _sc_mesh():
    return plsc.VectorSubcoreMesh(
        core_axis_name="c", subcore_axis_name="s",
        num_cores=NC, num_subcores=NS)


def _make_sc_agg(NPAD, D, nch, with_counts=False):
    rpt = NPAD // NS
    zfull, zrem = divmod(rpt, CHUNK)
    npair = nch // 2
    assert nch % 2 == 0

    out_type = [jax.ShapeDtypeStruct((NC, NPAD, D), jnp.float32)]
    scratch = [
        pltpu.VMEM((2, 4, CHUNK), jnp.int32),
        pltpu.VMEM((CHUNK, D), jnp.float32),
        pltpu.VMEM((CHUNK, D), jnp.float32),
        pltpu.VMEM_SHARED((NPAD, D), jnp.float32),
        pltpu.SemaphoreType.DMA,
        pltpu.SemaphoreType.DMA,
    ]
    if with_counts:
        out_type.append(jax.ShapeDtypeStruct((NC, NPAD, 16), jnp.float32))
        scratch += [
            pltpu.VMEM((CHUNK, 16), jnp.float32),
            pltpu.VMEM((CHUNK, 16), jnp.float32),
            pltpu.VMEM_SHARED((NPAD, 16), jnp.float32),
        ]

    @functools.partial(
        pl.kernel, mesh=_sc_mesh(),
        out_type=tuple(out_type) if with_counts else out_type[0],
        compiler_params=pltpu.CompilerParams(use_tc_tiling_on_sc=False),
        scratch_types=scratch)
    def k(feat, ed, sums, *rest):
        if with_counts:
            cnts, idx, rows0, rows1, acc, sem0, sem1, ones, zc, cac = rest
        else:
            idx, rows0, rows1, acc, sem0, sem1 = rest
        cid = lax.axis_index("c")
        sid = lax.axis_index("s")
        wid = cid * NS + sid

        zv = jnp.zeros((16,), jnp.float32)

        def init_row(r, carry):
            for j in range(D // 16):
                rows0[r, pl.ds(j * 16, 16)] = zv
            if with_counts:
                ones[r, :] = jnp.full((16,), 1.0, jnp.float32)
                zc[r, :] = zv
            return carry

        lax.fori_loop(0, CHUNK, init_row, 0)

        base = sid * rpt
        for kk in range(zfull):
            pltpu.sync_copy(rows0, acc.at[pl.ds(base + kk * CHUNK, CHUNK)])
            if with_counts:
                pltpu.sync_copy(zc, cac.at[pl.ds(base + kk * CHUNK, CHUNK)])
        if zrem:
            pltpu.sync_copy(rows0.at[pl.ds(0, zrem)],
                            acc.at[pl.ds(base + zfull * CHUNK, zrem)])
            if with_counts:
                pltpu.sync_copy(zc.at[pl.ds(0, zrem)],
                                cac.at[pl.ds(base + zfull * CHUNK, zrem)])
        plsc.subcore_barrier()

        pltpu.sync_copy(ed.at[wid, 0], idx.at[0])
        pltpu.async_copy(feat.at[idx.at[0, 0]], rows0, sem0)
        pltpu.async_copy(feat.at[idx.at[0, 2]], rows1, sem1)

        def step(j2, carry):
            cur = lax.rem(j2, 2)
            nxt = 1 - cur
            jn = jnp.minimum(j2 + 1, npair - 1)
            pltpu.sync_copy(ed.at[wid, jn], idx.at[nxt])
            pltpu.make_async_copy(feat.at[idx.at[cur, 0]], rows0, sem0).wait()
            pltpu.sync_copy(rows0, acc.at[idx.at[cur, 1]], add=True)
            if with_counts:
                pltpu.sync_copy(ones, cac.at[idx.at[cur, 1]], add=True)
            pltpu.async_copy(feat.at[idx.at[nxt, 0]], rows0, sem0)
            pltpu.make_async_copy(feat.at[idx.at[cur, 2]], rows1, sem1).wait()
            pltpu.sync_copy(rows1, acc.at[idx.at[cur, 3]], add=True)
            if with_counts:
                pltpu.sync_copy(ones, cac.at[idx.at[cur, 3]], add=True)
            pltpu.async_copy(feat.at[idx.at[nxt, 2]], rows1, sem1)
            return carry

        lax.fori_loop(0, npair, step, 0)
        pltpu.make_async_copy(feat.at[idx.at[0, 0]], rows0, sem0).wait()
        pltpu.make_async_copy(feat.at[idx.at[0, 2]], rows1, sem1).wait()
        plsc.subcore_barrier()

        pltpu.sync_copy(acc.at[pl.ds(base, rpt)],
                        sums.at[cid, pl.ds(base, rpt)])
        if with_counts:
            pltpu.sync_copy(cac.at[pl.ds(base, rpt)],
                            cnts.at[cid, pl.ds(base, rpt)])

    return k


def _make_sc_counts(NPAD, nch):
    rpt = NPAD // NS
    zfull, zrem = divmod(rpt, CHUNK)

    @functools.partial(
        pl.kernel, mesh=_sc_mesh(),
        out_type=jax.ShapeDtypeStruct((NC, NPAD, 16), jnp.float32),
        compiler_params=pltpu.CompilerParams(use_tc_tiling_on_sc=False),
        scratch_types=[
            pltpu.VMEM((nch, CHUNK), jnp.int32),
            pltpu.VMEM((CHUNK, 16), jnp.float32),
            pltpu.VMEM((CHUNK, 16), jnp.float32),
            pltpu.VMEM_SHARED((NPAD, 16), jnp.float32),
        ])
    def k(dstr, cnts, dst_i, ones, zc, cac):
        cid = lax.axis_index("c")
        sid = lax.axis_index("s")
        wid = cid * NS + sid

        def init_row(r, carry):
            ones[r, :] = jnp.full((16,), 1.0, jnp.float32)
            zc[r, :] = jnp.zeros((16,), jnp.float32)
            return carry

        lax.fori_loop(0, CHUNK, init_row, 0)

        base = sid * rpt
        for kk in range(zfull):
            pltpu.sync_copy(zc, cac.at[pl.ds(base + kk * CHUNK, CHUNK)])
        if zrem:
            pltpu.sync_copy(zc.at[pl.ds(0, zrem)],
                            cac.at[pl.ds(base + zfull * CHUNK, zrem)])
        pltpu.sync_copy(dstr.at[wid], dst_i)
        plsc.subcore_barrier()

        def step(j, carry):
            pltpu.sync_copy(ones, cac.at[dst_i.at[j]], add=True)
            return carry

        lax.fori_loop(0, nch, step, 0)
        plsc.subcore_barrier()

        pltpu.sync_copy(cac.at[pl.ds(base, rpt)],
                        cnts.at[cid, pl.ds(base, rpt)])

    return k


def _dense1_body(sum1, cnt, x, w1lT, b1, w1rT, w2lT, w2rT, b2, p2, r2):
    s = sum1[0] + sum1[1]
    c = cnt[0, :, 0:1] + cnt[1, :, 0:1]
    agg = s / jnp.maximum(c, 1.0)
    h = (jnp.dot(agg, w1lT[...], preferred_element_type=jnp.float32)
         + b1[...]
         + jnp.dot(x[...], w1rT[...], preferred_element_type=jnp.float32))
    p2[...] = jnp.dot(h, w2lT[...], preferred_element_type=jnp.float32)
    r2[...] = (jnp.dot(h, w2rT[...], preferred_element_type=jnp.float32)
               + b2[...])


def _dense2_body(sum2, cnt, r2, wlinT, blin, z_out, x_out):
    s = sum2[0] + sum2[1]
    c = cnt[0, :, 0:1] + cnt[1, :, 0:1]
    o = s / jnp.maximum(c, 1.0) + r2[...]
    nrm = jnp.sqrt(jnp.sum(o * o, axis=1, keepdims=True))
    z = o / jnp.maximum(nrm, 1e-12)
    z_out[...] = z
    t = jnp.dot(z, wlinT[...], preferred_element_type=jnp.float32) + blin[...]
    x_out[...] = jnp.where(t > 0, t, jnp.exp(t) - 1.0)


def _decode_body(zrow, zfull, out):
    p = lax.dot_general(zrow[...], zfull[...], (((1,), (1,)), ((), ())),
                        preferred_element_type=jnp.float32)
    out[...] = jax.nn.sigmoid(p)


def _dense1(N, sums1, cnts, x, w1lT, b1, w1rT, w2lT, w2rT, b2):
    BM = 1000
    grid = (N // BM,)
    full = lambda shape: pl.BlockSpec(shape, lambda i: (0,) * len(shape))
    return pl.pallas_call(
        _dense1_body,
        grid=grid,
        in_specs=[
            pl.BlockSpec((NC, BM, 128), lambda i: (0, i, 0)),
            pl.BlockSpec((NC, BM, 16), lambda i: (0, i, 0)),
            pl.BlockSpec((BM, 128), lambda i: (i, 0)),
            full((128, 128)), full((1, 128)), full((128, 128)),
            full((128, 64)), full((128, 64)), full((1, 64)),
        ],
        out_specs=[
            pl.BlockSpec((BM, 64), lambda i: (i, 0)),
            pl.BlockSpec((BM, 64), lambda i: (i, 0)),
        ],
        out_shape=[
            jax.ShapeDtypeStruct((N, 64), jnp.float32),
            jax.ShapeDtypeStruct((N, 64), jnp.float32),
        ],
    )(sums1, cnts, x, w1lT, b1, w1rT, w2lT, w2rT, b2)


def _dense2(N, sums2, cnts, r2, wlinT, blin):
    BM = 1000
    grid = (N // BM,)
    full = lambda shape: pl.BlockSpec(shape, lambda i: (0,) * len(shape))
    return pl.pallas_call(
        _dense2_body,
        grid=grid,
        in_specs=[
            pl.BlockSpec((NC, BM, 64), lambda i: (0, i, 0)),
            pl.BlockSpec((NC, BM, 16), lambda i: (0, i, 0)),
            pl.BlockSpec((BM, 64), lambda i: (i, 0)),
            full((64, 128)), full((1, 128)),
        ],
        out_specs=[
            pl.BlockSpec((BM, 64), lambda i: (i, 0)),
            pl.BlockSpec((BM, 128), lambda i: (i, 0)),
        ],
        out_shape=[
            jax.ShapeDtypeStruct((N, 64), jnp.float32),
            jax.ShapeDtypeStruct((N, 128), jnp.float32),
        ],
    )(sums2, cnts, r2, wlinT, blin)


def _decode(N, z):
    BM = 512
    grid = (pl.cdiv(N, BM),)
    return pl.pallas_call(
        _decode_body,
        grid=grid,
        in_specs=[
            pl.BlockSpec((BM, 64), lambda i: (i, 0)),
            pl.BlockSpec((N, 64), lambda i: (0, 0)),
        ],
        out_specs=pl.BlockSpec((BM, N), lambda i: (i, 0)),
        out_shape=jax.ShapeDtypeStruct((N, N), jnp.float32),
    )(z, z)


def kernel(x, edge_index, W1_l, b1_l, W1_r, W2_l, b2_l, W2_r, W_lin, b_lin):
    N, D_in = x.shape
    E = edge_index.shape[1]
    D_hid = W1_l.shape[0]
    D_out = W2_l.shape[0]
    ept = E // NT
    nch = ept // CHUNK
    NPAD = ((N + 8 * NS - 1) // (8 * NS)) * (8 * NS)

    src = edge_index[0].reshape(NT, nch, CHUNK)
    dst = edge_index[1].reshape(NT, nch, CHUNK)
    ed = jnp.stack([src[:, 0::2], dst[:, 0::2], src[:, 1::2], dst[:, 1::2]],
                   axis=2)

    agg1 = _make_sc_agg(NPAD, D_in, nch, with_counts=True)
    agg2 = _make_sc_agg(NPAD, D_out, nch)

    sums1, cnts = agg1(x, ed)
    p2, r2 = _dense1(N, sums1, cnts, x,
                     W1_l.T, b1_l.reshape(1, -1), W1_r.T,
                     W2_l.T, W2_r.T, b2_l.reshape(1, -1))
    sums2 = agg2(p2, ed)
    z, x_ = _dense2(N, sums2, cnts, r2, W_lin.T, b_lin.reshape(1, -1))
    A_pred = _decode(N, z)
    return (A_pred, z, x_)


# speedup vs baseline: 1.0128x; 1.0128x over previous
import functools

import jax
import jax.numpy as jnp
from jax import lax
from jax.experimental import pallas as pl
from jax.experimental.pallas import tpu as pltpu
from jax.experimental.pallas import tpu_sc as plsc

NC = 2
NS = 16
NT = NC * NS
CHUNK = 125


def _sc_mesh():
    return plsc.VectorSubcoreMesh(
        core_axis_name="c", subcore_axis_name="s",
        num_cores=NC, num_subcores=NS)


def _make_sc_agg(NPAD, D, nch):
    rpt = NPAD // NS
    zfull, zrem = divmod(rpt, CHUNK)
    npair = nch // 2
    assert nch % 2 == 0

    @functools.partial(
        pl.kernel, mesh=_sc_mesh(),
        out_type=jax.ShapeDtypeStruct((NC, NPAD, D), jnp.float32),
        compiler_params=pltpu.CompilerParams(use_tc_tiling_on_sc=False),
        scratch_types=[
            pltpu.VMEM((2, 4, CHUNK), jnp.int32),
            pltpu.VMEM((CHUNK, D), jnp.float32),
            pltpu.VMEM((CHUNK, D), jnp.float32),
            pltpu.VMEM_SHARED((NPAD, D), jnp.float32),
            pltpu.SemaphoreType.DMA,
            pltpu.SemaphoreType.DMA,
        ])
    def k(feat, ed, sums, idx, rows0, rows1, acc, sem0, sem1):
        cid = lax.axis_index("c")
        sid = lax.axis_index("s")
        wid = cid * NS + sid

        zv = jnp.zeros((16,), jnp.float32)

        def init_row(r, carry):
            for j in range(D // 16):
                rows0[r, pl.ds(j * 16, 16)] = zv
            return carry

        lax.fori_loop(0, CHUNK, init_row, 0)

        base = sid * rpt
        for kk in range(zfull):
            pltpu.sync_copy(rows0, acc.at[pl.ds(base + kk * CHUNK, CHUNK)])
        if zrem:
            pltpu.sync_copy(rows0.at[pl.ds(0, zrem)],
                            acc.at[pl.ds(base + zfull * CHUNK, zrem)])
        plsc.subcore_barrier()

        pltpu.sync_copy(ed.at[wid, 0], idx.at[0])
        pltpu.async_copy(feat.at[idx.at[0, 0]], rows0, sem0)
        pltpu.async_copy(feat.at[idx.at[0, 2]], rows1, sem1)

        def step(j2, carry):
            cur = lax.rem(j2, 2)
            nxt = 1 - cur
            jn = jnp.minimum(j2 + 1, npair - 1)
            pltpu.sync_copy(ed.at[wid, jn], idx.at[nxt])
            pltpu.make_async_copy(feat.at[idx.at[cur, 0]], rows0, sem0).wait()
            pltpu.sync_copy(rows0, acc.at[idx.at[cur, 1]], add=True)
            pltpu.async_copy(feat.at[idx.at[nxt, 0]], rows0, sem0)
            pltpu.make_async_copy(feat.at[idx.at[cur, 2]], rows1, sem1).wait()
            pltpu.sync_copy(rows1, acc.at[idx.at[cur, 3]], add=True)
            pltpu.async_copy(feat.at[idx.at[nxt, 2]], rows1, sem1)
            return carry

        lax.fori_loop(0, npair, step, 0)
        pltpu.make_async_copy(feat.at[idx.at[0, 0]], rows0, sem0).wait()
        pltpu.make_async_copy(feat.at[idx.at[0, 2]], rows1, sem1).wait()
        plsc.subcore_barrier()

        pltpu.sync_copy(acc.at[pl.ds(base, rpt)],
                        sums.at[cid, pl.ds(base, rpt)])

    return k


def _make_sc_counts(NPAD, nch):
    rpt = NPAD // NS
    zfull, zrem = divmod(rpt, CHUNK)

    @functools.partial(
        pl.kernel, mesh=_sc_mesh(),
        out_type=jax.ShapeDtypeStruct((NC, NPAD, 16), jnp.float32),
        compiler_params=pltpu.CompilerParams(use_tc_tiling_on_sc=False),
        scratch_types=[
            pltpu.VMEM((nch, CHUNK), jnp.int32),
            pltpu.VMEM((CHUNK, 16), jnp.float32),
            pltpu.VMEM((CHUNK, 16), jnp.float32),
            pltpu.VMEM_SHARED((NPAD, 16), jnp.float32),
        ])
    def k(dstr, cnts, dst_i, ones, zc, cac):
        cid = lax.axis_index("c")
        sid = lax.axis_index("s")
        wid = cid * NS + sid

        def init_row(r, carry):
            ones[r, :] = jnp.full((16,), 1.0, jnp.float32)
            zc[r, :] = jnp.zeros((16,), jnp.float32)
            return carry

        lax.fori_loop(0, CHUNK, init_row, 0)

        base = sid * rpt
        for kk in range(zfull):
            pltpu.sync_copy(zc, cac.at[pl.ds(base + kk * CHUNK, CHUNK)])
        if zrem:
            pltpu.sync_copy(zc.at[pl.ds(0, zrem)],
                            cac.at[pl.ds(base + zfull * CHUNK, zrem)])
        pltpu.sync_copy(dstr.at[wid], dst_i)
        plsc.subcore_barrier()

        def step(j, carry):
            pltpu.sync_copy(ones, cac.at[dst_i.at[j]], add=True)
            return carry

        lax.fori_loop(0, nch, step, 0)
        plsc.subcore_barrier()

        pltpu.sync_copy(cac.at[pl.ds(base, rpt)],
                        cnts.at[cid, pl.ds(base, rpt)])

    return k


def _dense1_body(sum1, cnt, x, w1lT, b1, w1rT, w2lT, w2rT, b2, p2, r2):
    s = sum1[0] + sum1[1]
    c = cnt[0, :, 0:1] + cnt[1, :, 0:1]
    agg = s / jnp.maximum(c, 1.0)
    h = (jnp.dot(agg, w1lT[...], preferred_element_type=jnp.float32)
         + b1[...]
         + jnp.dot(x[...], w1rT[...], preferred_element_type=jnp.float32))
    p2[...] = jnp.dot(h, w2lT[...], preferred_element_type=jnp.float32)
    r2[...] = (jnp.dot(h, w2rT[...], preferred_element_type=jnp.float32)
               + b2[...])


def _dense2_body(sum2, cnt, r2, wlinT, blin, z_out, x_out):
    s = sum2[0] + sum2[1]
    c = cnt[0, :, 0:1] + cnt[1, :, 0:1]
    o = s / jnp.maximum(c, 1.0) + r2[...]
    nrm = jnp.sqrt(jnp.sum(o * o, axis=1, keepdims=True))
    z = o / jnp.maximum(nrm, 1e-12)
    z_out[...] = z
    t = jnp.dot(z, wlinT[...], preferred_element_type=jnp.float32) + blin[...]
    x_out[...] = jnp.where(t > 0, t, jnp.exp(t) - 1.0)


def _decode_body(zrow, zfull, out):
    p = lax.dot_general(zrow[...], zfull[...], (((1,), (1,)), ((), ())),
                        preferred_element_type=jnp.float32)
    out[...] = jax.nn.sigmoid(p)


def _dense1(N, sums1, cnts, x, w1lT, b1, w1rT, w2lT, w2rT, b2):
    BM = 1000
    grid = (N // BM,)
    full = lambda shape: pl.BlockSpec(shape, lambda i: (0,) * len(shape))
    return pl.pallas_call(
        _dense1_body,
        grid=grid,
        in_specs=[
            pl.BlockSpec((NC, BM, 128), lambda i: (0, i, 0)),
            pl.BlockSpec((NC, BM, 16), lambda i: (0, i, 0)),
            pl.BlockSpec((BM, 128), lambda i: (i, 0)),
            full((128, 128)), full((1, 128)), full((128, 128)),
            full((128, 64)), full((128, 64)), full((1, 64)),
        ],
        out_specs=[
            pl.BlockSpec((BM, 64), lambda i: (i, 0)),
            pl.BlockSpec((BM, 64), lambda i: (i, 0)),
        ],
        out_shape=[
            jax.ShapeDtypeStruct((N, 64), jnp.float32),
            jax.ShapeDtypeStruct((N, 64), jnp.float32),
        ],
    )(sums1, cnts, x, w1lT, b1, w1rT, w2lT, w2rT, b2)


def _dense2(N, sums2, cnts, r2, wlinT, blin):
    BM = 1000
    grid = (N // BM,)
    full = lambda shape: pl.BlockSpec(shape, lambda i: (0,) * len(shape))
    return pl.pallas_call(
        _dense2_body,
        grid=grid,
        in_specs=[
            pl.BlockSpec((NC, BM, 64), lambda i: (0, i, 0)),
            pl.BlockSpec((NC, BM, 16), lambda i: (0, i, 0)),
            pl.BlockSpec((BM, 64), lambda i: (i, 0)),
            full((64, 128)), full((1, 128)),
        ],
        out_specs=[
            pl.BlockSpec((BM, 64), lambda i: (i, 0)),
            pl.BlockSpec((BM, 128), lambda i: (i, 0)),
        ],
        out_shape=[
            jax.ShapeDtypeStruct((N, 64), jnp.float32),
            jax.ShapeDtypeStruct((N, 128), jnp.float32),
        ],
    )(sums2, cnts, r2, wlinT, blin)


def _decode(N, z):
    BM = 512
    grid = (pl.cdiv(N, BM),)
    return pl.pallas_call(
        _decode_body,
        grid=grid,
        in_specs=[
            pl.BlockSpec((BM, 64), lambda i: (i, 0)),
            pl.BlockSpec((N, 64), lambda i: (0, 0)),
        ],
        out_specs=pl.BlockSpec((BM, N), lambda i: (i, 0)),
        out_shape=jax.ShapeDtypeStruct((N, N), jnp.float32),
    )(z, z)


def kernel(x, edge_index, W1_l, b1_l, W1_r, W2_l, b2_l, W2_r, W_lin, b_lin):
    N, D_in = x.shape
    E = edge_index.shape[1]
    D_hid = W1_l.shape[0]
    D_out = W2_l.shape[0]
    ept = E // NT
    nch = ept // CHUNK
    NPAD = ((N + 8 * NS - 1) // (8 * NS)) * (8 * NS)

    src = edge_index[0].reshape(NT, nch, CHUNK)
    dst = edge_index[1].reshape(NT, nch, CHUNK)
    ed = jnp.stack([src[:, 0::2], dst[:, 0::2], src[:, 1::2], dst[:, 1::2]],
                   axis=2)

    agg1 = _make_sc_agg(NPAD, D_in, nch)
    agg2 = _make_sc_agg(NPAD, D_out, nch)
    counts = _make_sc_counts(NPAD, nch)

    cnts = counts(dst)
    sums1 = agg1(x, ed)
    p2, r2 = _dense1(N, sums1, cnts, x,
                     W1_l.T, b1_l.reshape(1, -1), W1_r.T,
                     W2_l.T, W2_r.T, b2_l.reshape(1, -1))
    sums2 = agg2(p2, ed)
    z, x_ = _dense2(N, sums2, cnts, r2, W_lin.T, b_lin.reshape(1, -1))
    A_pred = _decode(N, z)
    return (A_pred, z, x_)
